# Initial kernel scaffold; baseline (speedup 1.0000x reference)
import jax
import jax.numpy as jnp
from jax.experimental import pallas as pl


def kernel(x, W, pe):
    raise NotImplementedError("write your pallas kernel here")


# speedup vs baseline: 1.0013x; 1.0013x over previous
import jax
import jax.numpy as jnp
from jax import lax
from jax.experimental import pallas as pl
from jax.experimental.pallas import tpu as pltpu
from jax.experimental.pallas import tpu_sc as plsc

B, L, D = 4, 8192, 1024
NC, NS = 2, 16
NW = NC * NS
P = L // NW
C = 8
STEPS = P // C
LANES = 16
DCH = D // LANES


def _body(x_hbm, W_hbm, pe_hbm, out_hbm, idx_v, pe_v, rows_v, sem):
    wid = lax.axis_index("s") * NC + lax.axis_index("c")
    base = wid * P

    for b in range(B):
        pltpu.sync_copy(x_hbm.at[b, pl.ds(base, P)], idx_v.at[b])

    def step(s, carry):
        l0 = base + s * C
        pltpu.sync_copy(pe_hbm.at[pl.ds(l0, C)], pe_v)
        for b in range(B):
            pltpu.async_copy(
                W_hbm.at[idx_v.at[b, pl.ds(s * C, C)]],
                rows_v.at[pl.ds(b * C, C)],
                sem,
            ).wait()

        def add_row(j, carry2):
            for d in range(DCH):
                pv = pe_v[j, pl.ds(d * LANES, LANES)]
                for b in range(B):
                    r = b * C + j
                    rows_v[r, pl.ds(d * LANES, LANES)] = (
                        rows_v[r, pl.ds(d * LANES, LANES)] + pv
                    )
            return carry2

        lax.fori_loop(0, C, add_row, 0)

        for b in range(B):
            pltpu.sync_copy(
                rows_v.at[pl.ds(b * C, C)], out_hbm.at[b, pl.ds(l0, C)]
            )
        return carry

    lax.fori_loop(0, STEPS, step, 0)


_emb = pl.kernel(
    _body,
    out_type=jax.ShapeDtypeStruct((B, L, D), jnp.float32),
    mesh=plsc.VectorSubcoreMesh(core_axis_name="c", subcore_axis_name="s"),
    scratch_types=[
        pltpu.VMEM((B, P), jnp.int32),
        pltpu.VMEM((C, D), jnp.float32),
        pltpu.VMEM((B * C, D), jnp.float32),
        pltpu.SemaphoreType.DMA,
    ],
)


def kernel(x, W, pe):
    return _emb(x, W, pe.reshape(L, D))


# speedup vs baseline: 1.6309x; 1.6288x over previous
import jax
import jax.numpy as jnp
from jax import lax
from jax.experimental import pallas as pl
from jax.experimental.pallas import tpu as pltpu
from jax.experimental.pallas import tpu_sc as plsc

B, L, D = 4, 8192, 1024
NC, NS = 2, 16
NW = NC * NS
P = L // NW
C = 8
STEPS = P // C
LANES = 16
DCH = D // LANES


def _body(x_hbm, W_hbm, pe_hbm, out_hbm, idx_v, pe_v, rows_v, lsem, ssem):
    wid = lax.axis_index("s") * NC + lax.axis_index("c")
    base = wid * P

    for b in range(B):
        pltpu.sync_copy(x_hbm.at[b, pl.ds(base, P)], idx_v.at[b])

    def issue_loads(s, q):
        l0 = base + s * C
        hs = [pltpu.async_copy(pe_hbm.at[pl.ds(l0, C)], pe_v.at[q], lsem.at[q])]
        for b in range(B):
            hs.append(
                pltpu.async_copy(
                    W_hbm.at[idx_v.at[b, pl.ds(s * C, C)]],
                    rows_v.at[q, pl.ds(b * C, C)],
                    lsem.at[q],
                )
            )
        return hs

    def compute(q):
        def add_row(j, c2):
            for d in range(DCH):
                pv = pe_v[q, j, pl.ds(d * LANES, LANES)]
                for b in range(B):
                    r = b * C + j
                    rows_v[q, r, pl.ds(d * LANES, LANES)] = (
                        rows_v[q, r, pl.ds(d * LANES, LANES)] + pv
                    )
            return c2

        lax.fori_loop(0, C, add_row, 0)

    def issue_stores(s, q):
        l0 = base + s * C
        for b in range(B):
            pltpu.async_copy(
                rows_v.at[q, pl.ds(b * C, C)],
                out_hbm.at[b, pl.ds(l0, C)],
                ssem.at[q],
            )

    def wait_stores(q):
        for b in range(B):
            pltpu.make_async_copy(
                rows_v.at[q, pl.ds(b * C, C)],
                out_hbm.at[b, pl.ds(base, C)],
                ssem.at[q],
            ).wait()

    def step(s, carry):
        q = jnp.bitwise_and(s, 1)
        p = 1 - q

        @pl.when(s >= 2)
        def _():
            wait_stores(q)

        hs = issue_loads(s, q)

        @pl.when(s >= 1)
        def _():
            compute(p)
            issue_stores(s - 1, p)

        for h in hs:
            h.wait()
        return carry

    lax.fori_loop(0, STEPS, step, 0)

    wait_stores(0)
    compute(1)
    l_last = base + (STEPS - 1) * C
    for b in range(B):
        pltpu.sync_copy(
            rows_v.at[1, pl.ds(b * C, C)], out_hbm.at[b, pl.ds(l_last, C)]
        )


_emb = pl.kernel(
    _body,
    out_type=jax.ShapeDtypeStruct((B, L, D), jnp.float32),
    mesh=plsc.VectorSubcoreMesh(core_axis_name="c", subcore_axis_name="s"),
    scratch_types=[
        pltpu.VMEM((B, P), jnp.int32),
        pltpu.VMEM((2, C, D), jnp.float32),
        pltpu.VMEM((2, B * C, D), jnp.float32),
        pltpu.SemaphoreType.DMA((2,)),
        pltpu.SemaphoreType.DMA((2,)),
    ],
)


def kernel(x, W, pe):
    return _emb(x, W, pe.reshape(L, D))


# speedup vs baseline: 1.7199x; 1.0545x over previous
import jax
import jax.numpy as jnp
from jax import lax
from jax.experimental import pallas as pl
from jax.experimental.pallas import tpu as pltpu
from jax.experimental.pallas import tpu_sc as plsc

B, L, D = 4, 8192, 1024
NC, NS = 2, 16
NW = NC * NS
P = L // NW
C = 8
STEPS = P // C
NBUF = 3
LANES = 16
DCH = D // LANES


def _body(x_hbm, W_hbm, pe_hbm, out_hbm, idx_v, pe_v, rows_v, lsem, ssem):
    wid = lax.axis_index("s") * NC + lax.axis_index("c")
    base = wid * P

    for b in range(B):
        pltpu.sync_copy(x_hbm.at[b, pl.ds(base, P)], idx_v.at[b])

    def issue_loads(s, q):
        l0 = base + s * C
        pltpu.async_copy(pe_hbm.at[pl.ds(l0, C)], pe_v.at[q], lsem.at[q])
        for b in range(B):
            pltpu.async_copy(
                W_hbm.at[idx_v.at[b, pl.ds(s * C, C)]],
                rows_v.at[q, pl.ds(b * C, C)],
                lsem.at[q],
            )

    def wait_loads(q):
        pltpu.make_async_copy(
            pe_hbm.at[pl.ds(base, C)], pe_v.at[q], lsem.at[q]
        ).wait()
        for b in range(B):
            pltpu.make_async_copy(
                W_hbm.at[idx_v.at[b, pl.ds(0, C)]],
                rows_v.at[q, pl.ds(b * C, C)],
                lsem.at[q],
            ).wait()

    def compute(q):
        def add_row(j, c2):
            for d in range(DCH):
                pv = pe_v[q, j, pl.ds(d * LANES, LANES)]
                for b in range(B):
                    r = b * C + j
                    rows_v[q, r, pl.ds(d * LANES, LANES)] = (
                        rows_v[q, r, pl.ds(d * LANES, LANES)] + pv
                    )
            return c2

        lax.fori_loop(0, C, add_row, 0)

    def issue_stores(s, q):
        l0 = base + s * C
        for b in range(B):
            pltpu.async_copy(
                rows_v.at[q, pl.ds(b * C, C)],
                out_hbm.at[b, pl.ds(l0, C)],
                ssem.at[q],
            )

    def wait_stores(q):
        for b in range(B):
            pltpu.make_async_copy(
                rows_v.at[q, pl.ds(b * C, C)],
                out_hbm.at[b, pl.ds(base, C)],
                ssem.at[q],
            ).wait()

    issue_loads(0, 0)
    issue_loads(1, 1)

    def step(s, carry):
        q = lax.rem(s, NBUF)
        qn = lax.rem(s + 2, NBUF)

        @pl.when(s + 2 < STEPS)
        def _():
            @pl.when(s >= 1)
            def _():
                wait_stores(qn)

            issue_loads(s + 2, qn)

        wait_loads(q)
        compute(q)
        issue_stores(s, q)
        return carry

    lax.fori_loop(0, STEPS, step, 0)

    wait_stores((STEPS - 2) % NBUF)
    wait_stores((STEPS - 1) % NBUF)


_emb = pl.kernel(
    _body,
    out_type=jax.ShapeDtypeStruct((B, L, D), jnp.float32),
    mesh=plsc.VectorSubcoreMesh(core_axis_name="c", subcore_axis_name="s"),
    scratch_types=[
        pltpu.VMEM((B, P), jnp.int32),
        pltpu.VMEM((NBUF, C, D), jnp.float32),
        pltpu.VMEM((NBUF, B * C, D), jnp.float32),
        pltpu.SemaphoreType.DMA((NBUF,)),
        pltpu.SemaphoreType.DMA((NBUF,)),
    ],
)


def kernel(x, W, pe):
    return _emb(x, W, pe.reshape(L, D))
